# Initial kernel scaffold; baseline (speedup 1.0000x reference)
import jax
import jax.numpy as jnp
from jax.experimental import pallas as pl


def kernel(curr_fea, edge_index, W_self, b_self, W_neig, b_neig):
    raise NotImplementedError("write your pallas kernel here")


# speedup vs baseline: 7.5108x; 7.5108x over previous
import functools

import jax
import jax.numpy as jnp
from jax import lax
from jax.experimental import pallas as pl
from jax.experimental.pallas import tpu as pltpu
from jax.experimental.pallas import tpu_sc as plsc

N_NODES = 10000
D = 128
N_EDGES = 320000

NC = 2
NS = 16
CH = 128

N_PAD = 10112
ROWS_PER_TILE = N_PAD // NS

E_ALL = 2 * N_EDGES
CHUNKS_PER_TILE = -(-E_ALL // (NC * NS * CH))
E_PAD = NC * NS * CH * CHUNKS_PER_TILE
EDGES_PER_TILE = CH * CHUNKS_PER_TILE
EDGES_PER_CORE = NS * EDGES_PER_TILE


def _sc_body(fea_hbm, src_hbm, dst_hbm, zeros_hbm, part_hbm,
             acc_sh, sidx_v, didx_v, rows_v, sem):
    c = lax.axis_index("c")
    s = lax.axis_index("s")

    row0 = s * ROWS_PER_TILE
    pltpu.sync_copy(zeros_hbm.at[pl.ds(row0, ROWS_PER_TILE)],
                    acc_sh.at[pl.ds(row0, ROWS_PER_TILE)])
    plsc.subcore_barrier()

    base = c * EDGES_PER_CORE + s * EDGES_PER_TILE

    def step(i, carry):
        off = base + i * CH
        pltpu.sync_copy(src_hbm.at[pl.ds(off, CH)], sidx_v)
        pltpu.sync_copy(dst_hbm.at[pl.ds(off, CH)], didx_v)
        pltpu.async_copy(fea_hbm.at[sidx_v], rows_v, sem).wait()
        pltpu.sync_copy(rows_v, acc_sh.at[didx_v], add=True)
        return carry

    lax.fori_loop(0, CHUNKS_PER_TILE, step, 0)
    plsc.subcore_barrier()

    pltpu.sync_copy(acc_sh.at[pl.ds(row0, ROWS_PER_TILE)],
                    part_hbm.at[c, pl.ds(row0, ROWS_PER_TILE)])


@jax.jit
def _sc_scatter(fea, src_all, dst_all, zeros):
    mesh = plsc.VectorSubcoreMesh(core_axis_name="c", subcore_axis_name="s")
    return pl.kernel(
        _sc_body,
        out_type=jax.ShapeDtypeStruct((NC, N_PAD, D), jnp.float32),
        mesh=mesh,
        scratch_types=[
            pltpu.VMEM_SHARED((N_PAD, D), jnp.float32),
            pltpu.VMEM((CH,), jnp.int32),
            pltpu.VMEM((CH,), jnp.int32),
            pltpu.VMEM((CH, D), jnp.float32),
            pltpu.SemaphoreType.DMA,
        ],
    )(fea, src_all, dst_all, zeros)


BLK = 2000


def _tc_body(fea_ref, p0_ref, p1_ref, ws_ref, wn_ref, bias_ref, out_ref):
    nei = p0_ref[...] + p1_ref[...]
    out_ref[...] = (
        jnp.dot(fea_ref[...], ws_ref[...], preferred_element_type=jnp.float32)
        + jnp.dot(nei, wn_ref[...], preferred_element_type=jnp.float32)
        + bias_ref[...]
    )


@jax.jit
def _tc_combine(fea, p0, p1, ws_t, wn_t, bias):
    grid = (N_NODES // BLK,)
    return pl.pallas_call(
        _tc_body,
        grid=grid,
        in_specs=[
            pl.BlockSpec((BLK, D), lambda i: (i, 0)),
            pl.BlockSpec((BLK, D), lambda i: (i, 0)),
            pl.BlockSpec((BLK, D), lambda i: (i, 0)),
            pl.BlockSpec((D, D), lambda i: (0, 0)),
            pl.BlockSpec((D, D), lambda i: (0, 0)),
            pl.BlockSpec((1, D), lambda i: (0, 0)),
        ],
        out_specs=pl.BlockSpec((BLK, D), lambda i: (i, 0)),
        out_shape=jax.ShapeDtypeStruct((N_NODES, D), jnp.float32),
    )(fea, p0, p1, ws_t, wn_t, bias)


def kernel(curr_fea, edge_index, W_self, b_self, W_neig, b_neig):
    src = edge_index[0].astype(jnp.int32)
    dst = edge_index[1].astype(jnp.int32)
    n_fill = E_PAD - E_ALL
    fill_src = jnp.zeros((n_fill,), jnp.int32)
    fill_dst = jnp.full((n_fill,), N_NODES, jnp.int32)
    src_all = jnp.concatenate([src, dst, fill_src])
    dst_all = jnp.concatenate([dst, src, fill_dst])
    zeros = jnp.zeros((N_PAD, D), jnp.float32)

    parts = _sc_scatter(curr_fea, src_all, dst_all, zeros)

    bias = (b_self + b_neig).reshape(1, D)
    return _tc_combine(curr_fea, parts[0, :N_NODES], parts[1, :N_NODES],
                       W_self.T, W_neig.T, bias)
